# baseline (speedup 1.0000x reference)
import jax
import jax.numpy as jnp
from jax import lax
from jax.experimental import pallas as pl

_DN = ('NCHW', 'OIHW', 'NCHW')


def _conv(x, w, b, stride):
    y = lax.conv_general_dilated(x, w, (stride, stride), [(1, 1), (1, 1)],
                                 dimension_numbers=_DN)
    return y + b[None, :, None, None]


def _vq_body(zf_ref, z_sq_ref, emb_ref, emb_sq_ref, q_ref):
    zf = zf_ref[...]
    emb = emb_ref[...]
    z_sq = z_sq_ref[...]
    emb_sq = emb_sq_ref[...]
    zdote = lax.dot_general(zf, emb, (((1,), (1,)), ((), ())),
                            preferred_element_type=jnp.float32)
    d = (z_sq - 2.0 * zdote) + emb_sq
    dmin = jnp.min(d, axis=1, keepdims=True)
    ids = lax.broadcasted_iota(jnp.int32, d.shape, 1)
    big = jnp.int32(d.shape[1] + 1)
    idx = jnp.min(jnp.where(d == dmin, ids, big), axis=1, keepdims=True)
    onehot = (ids == idx).astype(jnp.float32)
    q_ref[...] = jnp.dot(onehot, emb, preferred_element_type=jnp.float32)


def _quantize(z_flat, emb):
    r, dim = z_flat.shape
    k = emb.shape[0]
    blk = 224
    z_sq = (z_flat ** 2).sum(1, keepdims=True)
    emb_sq = (emb ** 2).sum(1)[None, :]
    return pl.pallas_call(
        _vq_body,
        grid=(r // blk,),
        in_specs=[
            pl.BlockSpec((blk, dim), lambda i: (i, 0)),
            pl.BlockSpec((blk, 1), lambda i: (i, 0)),
            pl.BlockSpec((k, dim), lambda i: (0, 0)),
            pl.BlockSpec((1, k), lambda i: (0, 0)),
        ],
        out_specs=pl.BlockSpec((blk, dim), lambda i: (i, 0)),
        out_shape=jax.ShapeDtypeStruct((r, dim), jnp.float32),
    )(z_flat, z_sq, emb, emb_sq)


_KTAPS = ((1, 3), (2, 0))
_DELTA = {0: 1, 1: 0, 2: 0, 3: -1}
_W = 14
_N = 196


def _phase_step(planes, wall, bias, ml, mr, lvl, cout, cpad, act):
    half = 2 ** lvl
    pall = [jnp.dot(wall, a, preferred_element_type=jnp.float32)
            for a in planes]
    out = []
    for yb_out in range(2 * half):
        r, yb = yb_out & 1, yb_out >> 1
        for xb_out in range(2 * half):
            c, xb = xb_out & 1, xb_out >> 1
            acc = None
            for ky in _KTAPS[r]:
                sy, ybm = divmod(yb + _DELTA[ky], half)
                for kx in _KTAPS[c]:
                    sx, xbm = divmod(xb + _DELTA[kx], half)
                    t = ky * 4 + kx
                    chunk = pall[ybm * half + xbm][t * cpad:t * cpad + cout]
                    s = sy * _W + sx
                    if s > 0:
                        chunk = jnp.concatenate(
                            [chunk[:, s:], jnp.zeros((cout, s), jnp.float32)],
                            1)
                    elif s < 0:
                        chunk = jnp.concatenate(
                            [jnp.zeros((cout, -s), jnp.float32), chunk[:, :s]],
                            1)
                    if sx == 1:
                        chunk = chunk * mr
                    elif sx == -1:
                        chunk = chunk * ml
                    acc = chunk if acc is None else acc + chunk
            out.append(act(acc + bias))
    return out


def _dec12_body(a_ref, w1_ref, b1_ref, w2_ref, b2_ref, ml_ref, mr_ref,
                o_ref):
    relu = jax.nn.relu
    ml = ml_ref[...]
    mr = mr_ref[...]
    planes = [a_ref[0, 0]]
    planes = _phase_step(planes, w1_ref[...], b1_ref[...], ml, mr,
                         0, 128, 128, relu)
    planes = _phase_step(planes, w2_ref[...], b2_ref[...], ml, mr,
                         1, 64, 64, relu)
    for p, arr in enumerate(planes):
        o_ref[0, p] = arr


def _dec34_body(a_ref, w3_ref, b3_ref, w4_ref, b4_ref, ml_ref, mr_ref,
                o_ref):
    relu = jax.nn.relu
    ml = ml_ref[...]
    mr = mr_ref[...]
    planes = [a_ref[0, p] for p in range(16)]
    planes = _phase_step(planes, w3_ref[...], b3_ref[...], ml, mr,
                         2, 32, 32, relu)
    planes = _phase_step(planes, w4_ref[...], b4_ref[...], ml, mr,
                         3, 3, 8, jnp.tanh)
    for p, arr in enumerate(planes):
        o_ref[0, p] = arr


def _wall(dw, cpad):
    cin, cout = dw.shape[0], dw.shape[1]
    wt = jnp.transpose(dw, (2, 3, 1, 0)).reshape(16, cout, cin)
    if cpad != cout:
        wt = jnp.pad(wt, ((0, 0), (0, cpad - cout), (0, 0)))
    return wt.reshape(16 * cpad, cin)


def _decode(qt, dws, dbs):
    n = _N
    w1, w2, w3, w4 = (_wall(dws[0], 128), _wall(dws[1], 64),
                      _wall(dws[2], 32), _wall(dws[3], 8))
    b1, b2, b3, b4 = (d[:, None] for d in dbs)
    xs = jnp.arange(n, dtype=jnp.int32) % _W
    ml = (xs != 0).astype(jnp.float32)[None, :]
    mr = (xs != _W - 1).astype(jnp.float32)[None, :]
    const = lambda i: (0, 0)
    mid = pl.pallas_call(
        _dec12_body,
        grid=(8,),
        in_specs=[
            pl.BlockSpec((1, 1, 512, n), lambda i: (i, 0, 0, 0)),
            pl.BlockSpec(w1.shape, const),
            pl.BlockSpec(b1.shape, const),
            pl.BlockSpec(w2.shape, const),
            pl.BlockSpec(b2.shape, const),
            pl.BlockSpec((1, n), const),
            pl.BlockSpec((1, n), const),
        ],
        out_specs=pl.BlockSpec((1, 16, 64, n), lambda i: (i, 0, 0, 0)),
        out_shape=jax.ShapeDtypeStruct((8, 16, 64, n), jnp.float32),
    )(qt, w1, b1, w2, b2, ml, mr)
    return pl.pallas_call(
        _dec34_body,
        grid=(8,),
        in_specs=[
            pl.BlockSpec((1, 16, 64, n), lambda i: (i, 0, 0, 0)),
            pl.BlockSpec(w3.shape, const),
            pl.BlockSpec(b3.shape, const),
            pl.BlockSpec(w4.shape, const),
            pl.BlockSpec(b4.shape, const),
            pl.BlockSpec((1, n), const),
            pl.BlockSpec((1, n), const),
        ],
        out_specs=pl.BlockSpec((1, 256, 3, n), lambda i: (i, 0, 0, 0)),
        out_shape=jax.ShapeDtypeStruct((8, 256, 3, n), jnp.float32),
    )(mid, w3, b3, w4, b4, ml, mr)


def kernel(x, w1, b1, w2, b2, w3, b3, w4, b4, emb,
           dw1, db1, dw2, db2, dw3, db3, dw4, db4):
    relu = jax.nn.relu
    z = relu(_conv(x, w1, b1, 2))
    z = relu(_conv(z, w2, b2, 2))
    z = relu(_conv(z, w3, b3, 2))
    z = relu(_conv(z, w4, b4, 2))
    zp = jnp.transpose(z, (0, 2, 3, 1))
    z_flat = zp.reshape(-1, emb.shape[1])
    q = _quantize(z_flat, emb)

    qt = jnp.transpose(q.reshape(8, _N, 512), (0, 2, 1)).reshape(8, 1, 512, _N)
    a = _decode(qt, (dw1, dw2, dw3, dw4), (db1, db2, db3, db4))
    y = a.reshape(8, 16, 16, 3, _W, _W).transpose(0, 3, 4, 1, 5, 2)
    return y.reshape(8, 3, 224, 224)


# speedup vs baseline: 1.0294x; 1.0294x over previous
import functools

import jax
import jax.numpy as jnp
from jax import lax
from jax.experimental import pallas as pl

_DN = ('NCHW', 'OIHW', 'NCHW')


def _conv(x, w, b, stride):
    y = lax.conv_general_dilated(x, w, (stride, stride), [(1, 1), (1, 1)],
                                 dimension_numbers=_DN)
    return y + b[None, :, None, None]


def _vq_body(zf_ref, z_sq_ref, emb_ref, emb_sq_ref, q_ref):
    zf = zf_ref[...]
    emb = emb_ref[...]
    z_sq = z_sq_ref[...]
    emb_sq = emb_sq_ref[...]
    zdote = lax.dot_general(zf, emb, (((1,), (1,)), ((), ())),
                            preferred_element_type=jnp.float32)
    d = (z_sq - 2.0 * zdote) + emb_sq
    dmin = jnp.min(d, axis=1, keepdims=True)
    ids = lax.broadcasted_iota(jnp.int32, d.shape, 1)
    big = jnp.int32(d.shape[1] + 1)
    idx = jnp.min(jnp.where(d == dmin, ids, big), axis=1, keepdims=True)
    onehot = (ids == idx).astype(jnp.float32)
    q_ref[...] = jnp.dot(onehot, emb, preferred_element_type=jnp.float32)


def _quantize(z_flat, emb):
    r, dim = z_flat.shape
    k = emb.shape[0]
    blk = 224
    z_sq = (z_flat ** 2).sum(1, keepdims=True)
    emb_sq = (emb ** 2).sum(1)[None, :]
    return pl.pallas_call(
        _vq_body,
        grid=(r // blk,),
        in_specs=[
            pl.BlockSpec((blk, dim), lambda i: (i, 0)),
            pl.BlockSpec((blk, 1), lambda i: (i, 0)),
            pl.BlockSpec((k, dim), lambda i: (0, 0)),
            pl.BlockSpec((1, k), lambda i: (0, 0)),
        ],
        out_specs=pl.BlockSpec((blk, dim), lambda i: (i, 0)),
        out_shape=jax.ShapeDtypeStruct((r, dim), jnp.float32),
    )(z_flat, z_sq, emb, emb_sq)


_KTAPS = ((1, 3), (2, 0))
_DELTA = {0: 1, 1: 0, 2: 0, 3: -1}
_W = 14
_N = 196


def _phase_body(a_ref, w_ref, b_ref, ml_ref, mr_ref, o_ref, *, lvl, cout,
                cpad, act):
    nplanes = 4 ** lvl
    half = 2 ** lvl
    bias = b_ref[...]
    ml = ml_ref[...]
    mr = mr_ref[...]
    wall = w_ref[...]
    pall = []
    for p in range(nplanes):
        a = a_ref[0, p]
        pall.append(jnp.dot(wall, a, preferred_element_type=jnp.float32))
    for yb_out in range(2 * half):
        r, yb = yb_out & 1, yb_out >> 1
        for xb_out in range(2 * half):
            c, xb = xb_out & 1, xb_out >> 1
            acc = None
            for ky in _KTAPS[r]:
                sy, ybm = divmod(yb + _DELTA[ky], half)
                for kx in _KTAPS[c]:
                    sx, xbm = divmod(xb + _DELTA[kx], half)
                    t = ky * 4 + kx
                    chunk = pall[ybm * half + xbm][t * cpad:t * cpad + cout]
                    s = sy * _W + sx
                    if s > 0:
                        chunk = jnp.concatenate(
                            [chunk[:, s:], jnp.zeros((cout, s), jnp.float32)],
                            1)
                    elif s < 0:
                        chunk = jnp.concatenate(
                            [jnp.zeros((cout, -s), jnp.float32), chunk[:, :s]],
                            1)
                    if sx == 1:
                        chunk = chunk * mr
                    elif sx == -1:
                        chunk = chunk * ml
                    acc = chunk if acc is None else acc + chunk
            o_ref[0, yb_out * 2 * half + xb_out] = act(acc + bias)


def _phase_layer(a, dw, db, lvl, act):
    b, nplanes, cin, n = a.shape
    cout = dw.shape[1]
    cpad = max(cout, 8)
    wt = jnp.transpose(dw, (2, 3, 1, 0)).reshape(16, cout, cin)
    if cpad != cout:
        wt = jnp.pad(wt, ((0, 0), (0, cpad - cout), (0, 0)))
    wall = wt.reshape(16 * cpad, cin)
    bias = db[:, None]
    xs = jnp.arange(n, dtype=jnp.int32) % _W
    ml = (xs != 0).astype(jnp.float32)[None, :]
    mr = (xs != _W - 1).astype(jnp.float32)[None, :]
    body = functools.partial(_phase_body, lvl=lvl, cout=cout, cpad=cpad,
                             act=act)
    return pl.pallas_call(
        body,
        grid=(b,),
        in_specs=[
            pl.BlockSpec((1, nplanes, cin, n), lambda i: (i, 0, 0, 0)),
            pl.BlockSpec((16 * cpad, cin), lambda i: (0, 0)),
            pl.BlockSpec((cout, 1), lambda i: (0, 0)),
            pl.BlockSpec((1, n), lambda i: (0, 0)),
            pl.BlockSpec((1, n), lambda i: (0, 0)),
        ],
        out_specs=pl.BlockSpec((1, 4 * nplanes, cout, n),
                               lambda i: (i, 0, 0, 0)),
        out_shape=jax.ShapeDtypeStruct((b, 4 * nplanes, cout, n),
                                       jnp.float32),
    )(a, wall, bias, ml, mr)


def kernel(x, w1, b1, w2, b2, w3, b3, w4, b4, emb,
           dw1, db1, dw2, db2, dw3, db3, dw4, db4):
    relu = jax.nn.relu
    z = relu(_conv(x, w1, b1, 2))
    z = relu(_conv(z, w2, b2, 2))
    z = relu(_conv(z, w3, b3, 2))
    z = relu(_conv(z, w4, b4, 2))
    zp = jnp.transpose(z, (0, 2, 3, 1))
    z_flat = zp.reshape(-1, emb.shape[1])
    q = _quantize(z_flat, emb)

    a = jnp.transpose(q.reshape(8, _N, 512), (0, 2, 1)).reshape(8, 1, 512, _N)
    a = _phase_layer(a, dw1, db1, 0, relu)
    a = _phase_layer(a, dw2, db2, 1, relu)
    a = _phase_layer(a, dw3, db3, 2, relu)
    a = _phase_layer(a, dw4, db4, 3, jnp.tanh)
    y = a.reshape(8, 16, 16, 3, _W, _W).transpose(0, 3, 4, 1, 5, 2)
    return y.reshape(8, 3, 224, 224)
